# baseline (speedup 1.0000x reference)
import jax
import jax.numpy as jnp
from jax.experimental import pallas as pl
from jax.experimental.pallas import tpu as pltpu

_ROWS = 128
_COLS = 100000
_SUP_STRIDE = 200
_SUP_LIMIT = 6400
_N_CHUNKS = 16
_CHUNK_ROWS = _ROWS // _N_CHUNKS
_PREFETCH = 4


def _chunk_slice(x_hbm, i):
    return x_hbm.at[pl.ds(i * _CHUNK_ROWS, _CHUNK_ROWS), :]


def _body(x_hbm, o_hbm, bufs, sem_in, sem_out):
    def start_in(i):
        pltpu.make_async_copy(
            _chunk_slice(x_hbm, i), bufs.at[i], sem_in.at[i]
        ).start()

    for j in range(_PREFETCH):
        start_in(j)
    neg = jnp.full((_CHUNK_ROWS, 1), -jnp.inf, jnp.float32)
    for i in range(_N_CHUNKS):
        pltpu.make_async_copy(
            _chunk_slice(x_hbm, i), bufs.at[i], sem_in.at[i]
        ).wait()
        for c in range(0, _SUP_LIMIT, _SUP_STRIDE):
            bufs[i, :, c : c + 1] = neg
        pltpu.make_async_copy(
            bufs.at[i], _chunk_slice(o_hbm, i), sem_out.at[i]
        ).start()
        if i + _PREFETCH < _N_CHUNKS:
            start_in(i + _PREFETCH)
    for i in range(_N_CHUNKS):
        pltpu.make_async_copy(
            bufs.at[i], _chunk_slice(o_hbm, i), sem_out.at[i]
        ).wait()


def kernel(scores):
    return pl.pallas_call(
        _body,
        in_specs=[pl.BlockSpec(memory_space=pl.MemorySpace.ANY)],
        out_specs=pl.BlockSpec(memory_space=pl.MemorySpace.ANY),
        out_shape=jax.ShapeDtypeStruct((_ROWS, _COLS), scores.dtype),
        scratch_shapes=[
            pltpu.MemorySpace.VMEM((_N_CHUNKS, _CHUNK_ROWS, _COLS), jnp.float32),
            pltpu.SemaphoreType.DMA((_N_CHUNKS,)),
            pltpu.SemaphoreType.DMA((_N_CHUNKS,)),
        ],
    )(scores)
